# baseline (speedup 1.0000x reference)
import functools

import jax
import jax.numpy as jnp
from jax import lax
from jax.experimental import pallas as pl
from jax.experimental.pallas import tpu as pltpu
from jax.experimental.pallas import tpu_sc as plsc

TOP_K = 8


def _routing_body(cls_ref, key_ref, ids_ref, rs_ref):
    eps = 1e-12
    k = key_ref[...]
    kn = jnp.sqrt(jnp.sum(k * k, axis=1, keepdims=True))
    k_n = k / jnp.maximum(kn, eps)
    x = cls_ref[...]
    xn = jnp.sqrt(jnp.sum(x * x, axis=1, keepdims=True))
    x_n = x / jnp.maximum(xn, eps)
    sim0 = jax.lax.dot_general(
        x_n, k_n, (((1,), (1,)), ((), ())),
        preferred_element_type=jnp.float32)
    B, P = sim0.shape

    col = jax.lax.broadcasted_iota(jnp.int32, (B, P), 1)
    sim = sim0
    counts2d = jnp.zeros((B, P), jnp.int32)
    for _ in range(TOP_K):
        m = jnp.max(sim, axis=1, keepdims=True)
        cand = jnp.where(sim == m, col, P)
        j = jnp.min(cand, axis=1, keepdims=True)
        oh = col == j
        counts2d = counts2d + oh.astype(jnp.int32)
        sim = jnp.where(oh, -jnp.inf, sim)

    cnt = jnp.sum(counts2d, axis=0, keepdims=True)
    p_row = jax.lax.broadcasted_iota(jnp.int32, (1, P), 1)
    key2 = (cnt * (2 * P) + (P - 1 - p_row)).astype(jnp.float32)
    ri = jax.lax.broadcasted_iota(jnp.int32, (P, P), 0)
    ci = jax.lax.broadcasted_iota(jnp.int32, (P, P), 1)
    ident = (ri == ci).astype(jnp.float32)
    key2_col = jax.lax.dot_general(
        ident, key2, (((1,), (1,)), ((), ())),
        preferred_element_type=jnp.float32)
    gt = (key2_col > key2).astype(jnp.int32)
    rank = jnp.sum(gt, axis=0, keepdims=True)
    for j in range(TOP_K):
        ids_ref[0, j] = jnp.sum(jnp.where(rank == j, p_row, 0))
    colsum = jnp.sum(sim0, axis=0, keepdims=True)
    sel = (rank < TOP_K).astype(jnp.float32)
    rs_ref[0, 0] = jnp.sum(colsum * sel) / B


def _routing(cls_features, prompt_key):
    return pl.pallas_call(
        _routing_body,
        out_shape=(
            jax.ShapeDtypeStruct((1, TOP_K), jnp.int32),
            jax.ShapeDtypeStruct((1, 1), jnp.float32),
        ),
        in_specs=[
            pl.BlockSpec(memory_space=pltpu.VMEM),
            pl.BlockSpec(memory_space=pltpu.VMEM),
        ],
        out_specs=(
            pl.BlockSpec(memory_space=pltpu.SMEM),
            pl.BlockSpec(memory_space=pltpu.SMEM),
        ),
    )(cls_features, prompt_key)


def _make_sc_gather(L, B, LEN, C, n_workers, pairs):
    D = LEN * C
    info = plsc.get_sparse_core_info()
    nc = info.num_cores
    mesh = plsc.VectorSubcoreMesh(core_axis_name="c", subcore_axis_name="s")

    @functools.partial(
        pl.kernel,
        mesh=mesh,
        out_type=jax.ShapeDtypeStruct((L * B * TOP_K, D), jnp.float32),
        scratch_types=[
            pltpu.VMEM((32,), jnp.int32),
            [pltpu.VMEM((1, D), jnp.float32) for _ in range(pairs)],
            pltpu.SemaphoreType.DMA,
            pltpu.SemaphoreType.DMA,
        ],
    )
    def sc_gather(table_hbm, idx_hbm, out_hbm, idx_v, rows_v, rsem, wsem):
        wid = lax.axis_index("s") * nc + lax.axis_index("c")
        pltpu.sync_copy(idx_hbm.at[wid], idx_v)

        gathers = []
        for j in range(pairs):
            gathers.append(pltpu.async_copy(
                table_hbm.at[idx_v.at[pl.ds(8 * j, 1)]],
                rows_v[j], rsem))
        for g in gathers:
            g.wait()

        def row_of(j, b):
            q = wid * pairs + j
            layer = q // TOP_K
            kk = q % TOP_K
            return layer * (B * TOP_K) + kk + TOP_K * b

        def fire(j, b):
            pltpu.async_copy(
                rows_v[j], out_hbm.at[pl.ds(row_of(j, b), 1)], wsem)

        def drain(j, b):
            pltpu.make_async_copy(
                rows_v[j], out_hbm.at[pl.ds(row_of(j, b), 1)], wsem).wait()

        def body(b, carry):
            for j in range(pairs):
                fire(j, b)

            @pl.when(b >= 2)
            def _():
                for j in range(pairs):
                    drain(j, b - 2)
            return carry

        lax.fori_loop(0, B, body, 0)
        for b in (B - 2, B - 1):
            for j in range(pairs):
                drain(j, b)

    return sc_gather


def kernel(x_embed, cls_features, prompt, prompt_key):
    B = x_embed.shape[0]
    L, P, LEN, C = prompt.shape
    n_workers = 32
    pairs = (L * TOP_K) // n_workers

    ids, rs = _routing(cls_features, prompt_key)

    w = jnp.arange(n_workers, dtype=jnp.int32)[:, None]
    jcol = jnp.arange(32, dtype=jnp.int32)[None, :]
    q = w * pairs + jcol // 8
    valid = (jcol % 8 == 0) & (jcol // 8 < pairs)
    qc = jnp.minimum(q, L * TOP_K - 1)
    rows = qc // TOP_K * P + ids[0, qc % TOP_K]
    idx16 = jnp.where(valid, rows, 0).astype(jnp.int32)

    table = prompt.reshape(L * P, LEN * C)
    sc_gather = _make_sc_gather(L, B, LEN, C, n_workers, pairs)
    out_flat = sc_gather(table, idx16)
    batched_prompt = out_flat.reshape(L, B, TOP_K * LEN, C)

    return batched_prompt, rs.reshape(())
